# baseline (speedup 1.0000x reference)
import functools

import jax
import jax.numpy as jnp
from jax import lax
from jax.experimental import pallas as pl
from jax.experimental.pallas import tpu as pltpu
from jax.experimental.pallas import tpu_sc as plsc


_FIX = 32767.0


def _trig_body(phase_ref, packed_ref):
    p = phase_ref[...]
    c = jnp.round(jnp.cos(p) * _FIX).astype(jnp.int32)
    s = jnp.round(jnp.sin(p) * _FIX).astype(jnp.int32)
    packed_ref[...] = (c & 0xFFFF) | (s << 16)


def _rel_trig_packed(rel_phase):
    r, d = rel_phase.shape
    return pl.pallas_call(
        _trig_body,
        out_shape=jax.ShapeDtypeStruct((r, d), jnp.int32),
    )(rel_phase)


_LANES = 16


def _make_sc_kernel(batch, dim, chunk, nbuf, dist):
    info = plsc.get_sparse_core_info()
    nc, ns = info.num_cores, info.num_subcores
    nw = nc * ns
    assert batch % (nw * chunk) == 0
    assert dist < nbuf
    bpw = batch // nw
    n_chunks = bpw // chunk
    mesh = plsc.VectorSubcoreMesh(core_axis_name="c", subcore_axis_name="s")

    f32 = jnp.float32
    out_sds = jax.ShapeDtypeStruct((batch, dim), f32)
    rows = lambda dt: pltpu.VMEM((chunk, dim), dt)
    inv_fix = jnp.float32(1.0 / _FIX)

    @functools.partial(
        pl.kernel,
        out_type=(out_sds, out_sds, out_sds, out_sds),
        mesh=mesh,
        scratch_types=[
            pltpu.VMEM((2 * bpw,), jnp.int32),
            pltpu.VMEM((bpw,), jnp.int32),
            [pltpu.VMEM((2 * chunk, dim), f32)
             for _ in range(nbuf)],
            [pltpu.VMEM((2 * chunk, dim), f32)
             for _ in range(nbuf)],
            [rows(jnp.int32) for _ in range(nbuf)],
            [pltpu.SemaphoreType.DMA for _ in range(nbuf)],
            [pltpu.SemaphoreType.DMA for _ in range(nbuf)],
            pltpu.SemaphoreType.DMA,
        ],
    )
    def sc_kernel(h_idx, r_idx, t_idx, ent_re, ent_im, trig_t,
                  hr_re_o, hr_im_o, t_re_o, t_im_o,
                  htidx_v, ridx_v, re_v, im_v, pk_v,
                  gsem, wsem, isem):
        wid = lax.axis_index("s") * nc + lax.axis_index("c")
        base = wid * bpw
        idx_cps = [pltpu.async_copy(r_idx.at[pl.ds(base, bpw)], ridx_v, isem)]
        for k in range(n_chunks):
            hsl = pl.ds(base + k * chunk, chunk)
            idx_cps.append(pltpu.async_copy(
                h_idx.at[hsl], htidx_v.at[pl.ds(2 * k * chunk, chunk)], isem))
            idx_cps.append(pltpu.async_copy(
                t_idx.at[hsl],
                htidx_v.at[pl.ds((2 * k + 1) * chunk, chunk)], isem))
        for d in idx_cps:
            d.wait()

        gd, wd = {}, {}

        def issue_gathers(cki):
            s = cki % nbuf
            il = htidx_v.at[pl.ds(cki * 2 * chunk, 2 * chunk)]
            ri = ridx_v.at[pl.ds(cki * chunk, chunk)]
            gd[s] = [
                pltpu.async_copy(ent_re.at[il], re_v[s], gsem[s]),
                pltpu.async_copy(ent_im.at[il], im_v[s], gsem[s]),
                pltpu.async_copy(trig_t.at[ri], pk_v[s], gsem[s]),
            ]

        for g in range(min(dist, n_chunks)):
            issue_gathers(g)
        for cki in range(n_chunks):
            g = cki + dist
            if g < n_chunks:
                so = g % nbuf
                if so in wd:
                    for d in wd.pop(so):
                        d.wait()
                issue_gathers(g)

            s = cki % nbuf
            sl = pl.ds(base + cki * chunk, chunk)
            for d in gd.pop(s):
                d.wait()
            tpart = pl.ds(chunk, chunk)
            wr = [
                pltpu.async_copy(re_v[s].at[tpart], t_re_o.at[sl], wsem[s]),
                pltpu.async_copy(im_v[s].at[tpart], t_im_o.at[sl], wsem[s]),
            ]

            hre, him, pk = re_v[s], im_v[s], pk_v[s]

            def row_body(r, carry):
                for j in range(dim // _LANES):
                    cs = pl.ds(j * _LANES, _LANES)
                    a = hre[r, cs]
                    b = him[r, cs]
                    x = pk[r, cs]
                    c = lax.shift_right_arithmetic(
                        lax.shift_left(x, 16), 16).astype(f32)
                    si = lax.shift_right_arithmetic(x, 16).astype(f32)
                    hre[r, cs] = (a * c - b * si) * inv_fix
                    him[r, cs] = (a * si + b * c) * inv_fix
                return carry

            lax.fori_loop(0, chunk, row_body, 0)

            hpart = pl.ds(0, chunk)
            wd[s] = wr + [
                pltpu.async_copy(re_v[s].at[hpart], hr_re_o.at[sl], wsem[s]),
                pltpu.async_copy(im_v[s].at[hpart], hr_im_o.at[sl], wsem[s]),
            ]

        for s in list(wd):
            for d in wd.pop(s):
                d.wait()

    return sc_kernel


@jax.jit
def kernel(h_idx, r_idx, t_idx, ent_re, ent_im, rel_phase):
    batch = h_idx.shape[0]
    dim = ent_re.shape[1]
    trig_t = _rel_trig_packed(rel_phase)
    sc = _make_sc_kernel(batch, dim, chunk=64, nbuf=3, dist=2)
    return sc(h_idx.astype(jnp.int32), r_idx.astype(jnp.int32),
              t_idx.astype(jnp.int32), ent_re, ent_im, trig_t)
